# baseline (speedup 1.0000x reference)
import jax
import jax.numpy as jnp
from jax import lax
from jax.experimental import pallas as pl
from jax.experimental.pallas import tpu as pltpu
from jax.experimental.pallas import tpu_sc as plsc

B = 4096
F = 26
V = 100000
D = 16
H = 32
FD = F * D

NC = 2
NS = 16
NW = NC * NS
BF = B * F
CHUNK = BF // NW
GSZ = 128
K = CHUNK // GSZ
LINES = F * V // 8


def _gather_body(lines_hbm, sub_hbm, tab_hbm, out_hbm,
                 lid_v, sub_v, buf_a, buf_b, out_v, sem_a, sem_b):
    wid = lax.axis_index("s") * NC + lax.axis_index("c")
    pltpu.sync_copy(lines_hbm.at[wid], lid_v)
    pltpu.sync_copy(sub_hbm.at[wid], sub_v)

    bufs = ((buf_a, sem_a), (buf_b, sem_b))

    def start(c):
        buf, sem = bufs[c % 2]
        return pltpu.async_copy(tab_hbm.at[lid_v.at[c]], buf, sem)

    pending = start(0)
    for c in range(K):
        cur_buf, _ = bufs[c % 2]
        this = pending
        if c + 1 < K:
            pending = start(c + 1)
        this.wait()

        def ext(g, _, c=c, cur_buf=cur_buf):
            j0 = g * 16
            s16vec = sub_v[c, pl.ds(j0, 16)]
            iota = lax.iota(jnp.int32, 16)
            for l in range(16):
                row = jnp.full((16,), j0 + l, dtype=jnp.int32)
                col = s16vec[l] + iota
                vals = plsc.load_gather(cur_buf, [row, col])
                out_v[c * 16 + 2 * g + (l // 8), pl.ds((l % 8) * 16, 16)] = vals
            return 0

        lax.fori_loop(0, GSZ // 16, ext, 0)

    pltpu.sync_copy(out_v, out_hbm.at[wid])


_SC_GATHER_CACHE = []


def _sc_gather_kernel():
    if not _SC_GATHER_CACHE:
        _SC_GATHER_CACHE.append(pl.kernel(
            _gather_body,
            out_type=jax.ShapeDtypeStruct((NW, CHUNK * D // 128, 128),
                                          jnp.float32),
            mesh=plsc.VectorSubcoreMesh(core_axis_name="c",
                                        subcore_axis_name="s",
                                        num_cores=NC, num_subcores=NS),
            scratch_types=[
                pltpu.VMEM((K, GSZ), jnp.int32),
                pltpu.VMEM((K, GSZ), jnp.int32),
                pltpu.VMEM((GSZ, 128), jnp.float32),
                pltpu.VMEM((GSZ, 128), jnp.float32),
                pltpu.VMEM((CHUNK * D // 128, 128), jnp.float32),
                pltpu.SemaphoreType.DMA,
                pltpu.SemaphoreType.DMA,
            ],
            compiler_params=pltpu.CompilerParams(needs_layout_passes=False, use_tc_tiling_on_sc=False),
        ))
    return _SC_GATHER_CACHE[0]


BLK = 512


def _tc_body(emb_ref, xv_ref, exp_ref, w1_ref, w2_ref, l1w_ref, l1b_ref,
             l2w_ref, l2b_ref, lw_ref, lb_ref, out_ref):
    xv_rep = jnp.dot(xv_ref[:], exp_ref[:], preferred_element_type=jnp.float32)
    scaled = emb_ref[:] * xv_rep
    first = jnp.dot(scaled, w1_ref[:], preferred_element_type=jnp.float32)
    s = jnp.dot(scaled, w2_ref[:], preferred_element_type=jnp.float32)
    x = first + s * s
    x = jnp.maximum(jnp.dot(x, l1w_ref[:], preferred_element_type=jnp.float32) + l1b_ref[:], 0.0)
    x = jnp.maximum(jnp.dot(x, l2w_ref[:], preferred_element_type=jnp.float32) + l2b_ref[:], 0.0)
    out_ref[:] = jnp.dot(x, lw_ref[:], preferred_element_type=jnp.float32) + lb_ref[:]


_tc_dense = pl.pallas_call(
    _tc_body,
    grid=(B // BLK,),
    in_specs=[
        pl.BlockSpec((BLK, FD), lambda i: (i, 0)),
        pl.BlockSpec((BLK, F), lambda i: (i, 0)),
        pl.BlockSpec((F, FD), lambda i: (0, 0)),
        pl.BlockSpec((FD, H), lambda i: (0, 0)),
        pl.BlockSpec((FD, H), lambda i: (0, 0)),
        pl.BlockSpec((H, H), lambda i: (0, 0)),
        pl.BlockSpec((1, H), lambda i: (0, 0)),
        pl.BlockSpec((H, H), lambda i: (0, 0)),
        pl.BlockSpec((1, H), lambda i: (0, 0)),
        pl.BlockSpec((H, 1), lambda i: (0, 0)),
        pl.BlockSpec((1, 1), lambda i: (0, 0)),
    ],
    out_specs=pl.BlockSpec((BLK, 1), lambda i: (i, 0)),
    out_shape=jax.ShapeDtypeStruct((B, 1), jnp.float32),
)


def kernel(Xi, Xv, tables, W1, W2, L1_w, L1_b, L2_w, L2_b, last_w, last_b):
    r = (Xi[:, :, 0].astype(jnp.int32)
         + jnp.arange(F, dtype=jnp.int32)[None, :] * V)
    lines = (r >> 3).reshape(NW, K, GSZ)
    sub16 = ((r & 7) << 4).reshape(NW, K, GSZ)
    tab2 = tables.reshape(LINES, 128)

    emb = _sc_gather_kernel()(lines, sub16, tab2)
    emb_flat = emb.reshape(B, FD)

    expmat = (jnp.arange(FD, dtype=jnp.int32)[None, :] // D
              == jnp.arange(F, dtype=jnp.int32)[:, None]).astype(jnp.float32)
    w1r = W1.reshape(H, FD).T
    w2r = W2.reshape(H, FD).T

    out = _tc_dense(
        emb_flat, Xv, expmat, w1r, w2r,
        L1_w.T, L1_b.reshape(1, H),
        L2_w.T, L2_b.reshape(1, H),
        last_w.T, last_b.reshape(1, 1),
    )
    return out[:, 0]


# speedup vs baseline: 1.0274x; 1.0274x over previous
import jax
import jax.numpy as jnp
from jax import lax
from jax.experimental import pallas as pl
from jax.experimental.pallas import tpu as pltpu
from jax.experimental.pallas import tpu_sc as plsc

B = 4096
F = 26
V = 100000
D = 16
H = 32
FD = F * D

NC = 2
NS = 16
NW = NC * NS
BF = B * F
CHUNK = BF // NW
GSZ = 128
K = CHUNK // GSZ


def _gather_body(idx_hbm, table_hbm, out_hbm, idx_v, rows_v, sem):
    wid = lax.axis_index("s") * NC + lax.axis_index("c")
    pltpu.sync_copy(idx_hbm.at[wid], idx_v)
    copies = []
    for j in range(K):
        copies.append(
            pltpu.async_copy(
                table_hbm.at[idx_v.at[j]],
                rows_v.at[pl.ds(j * GSZ, GSZ)],
                sem,
            )
        )
    for c in copies:
        c.wait()
    pltpu.sync_copy(rows_v, out_hbm.at[wid])


_SC_GATHER_CACHE = []


def _sc_gather_kernel():
    if not _SC_GATHER_CACHE:
        _SC_GATHER_CACHE.append(pl.kernel(
            _gather_body,
            out_type=jax.ShapeDtypeStruct((NW, CHUNK, D), jnp.float32),
            mesh=plsc.VectorSubcoreMesh(core_axis_name="c",
                                        subcore_axis_name="s",
                                        num_cores=NC, num_subcores=NS),
            scratch_types=[
                pltpu.VMEM((K, GSZ), jnp.int32),
                pltpu.VMEM((CHUNK, D), jnp.float32),
                pltpu.SemaphoreType.DMA,
            ],
            compiler_params=pltpu.CompilerParams(use_tc_tiling_on_sc=False),
        ))
    return _SC_GATHER_CACHE[0]


BLK = 512


def _tc_body(emb_ref, xv_ref, exp_ref, w1_ref, w2_ref, l1w_ref, l1b_ref,
             l2w_ref, l2b_ref, lw_ref, lb_ref, out_ref):
    xv_rep = jnp.dot(xv_ref[:], exp_ref[:], preferred_element_type=jnp.float32)
    scaled = emb_ref[:] * xv_rep
    first = jnp.dot(scaled, w1_ref[:], preferred_element_type=jnp.float32)
    s = jnp.dot(scaled, w2_ref[:], preferred_element_type=jnp.float32)
    x = first + s * s
    x = jnp.maximum(jnp.dot(x, l1w_ref[:], preferred_element_type=jnp.float32) + l1b_ref[:], 0.0)
    x = jnp.maximum(jnp.dot(x, l2w_ref[:], preferred_element_type=jnp.float32) + l2b_ref[:], 0.0)
    out_ref[:] = jnp.dot(x, lw_ref[:], preferred_element_type=jnp.float32) + lb_ref[:]


_tc_dense = pl.pallas_call(
    _tc_body,
    grid=(B // BLK,),
    in_specs=[
        pl.BlockSpec((BLK, FD), lambda i: (i, 0)),
        pl.BlockSpec((BLK, F), lambda i: (i, 0)),
        pl.BlockSpec((F, FD), lambda i: (0, 0)),
        pl.BlockSpec((FD, H), lambda i: (0, 0)),
        pl.BlockSpec((FD, H), lambda i: (0, 0)),
        pl.BlockSpec((H, H), lambda i: (0, 0)),
        pl.BlockSpec((1, H), lambda i: (0, 0)),
        pl.BlockSpec((H, H), lambda i: (0, 0)),
        pl.BlockSpec((1, H), lambda i: (0, 0)),
        pl.BlockSpec((H, 1), lambda i: (0, 0)),
        pl.BlockSpec((1, 1), lambda i: (0, 0)),
    ],
    out_specs=pl.BlockSpec((BLK, 1), lambda i: (i, 0)),
    out_shape=jax.ShapeDtypeStruct((B, 1), jnp.float32),
)


def kernel(Xi, Xv, tables, W1, W2, L1_w, L1_b, L2_w, L2_b, last_w, last_b):
    idx_flat = (Xi[:, :, 0].astype(jnp.int32)
                + jnp.arange(F, dtype=jnp.int32)[None, :] * V)
    idx2 = idx_flat.reshape(NW, K, GSZ)
    tables_flat = tables.reshape(F * V, D)

    emb = _sc_gather_kernel()(idx2, tables_flat)
    emb_flat = emb.reshape(B, FD)

    expmat = (jnp.arange(FD, dtype=jnp.int32)[None, :] // D
              == jnp.arange(F, dtype=jnp.int32)[:, None]).astype(jnp.float32)
    w1r = W1.reshape(H, FD).T
    w2r = W2.reshape(H, FD).T

    out = _tc_dense(
        emb_flat, Xv, expmat, w1r, w2r,
        L1_w.T, L1_b.reshape(1, H),
        L2_w.T, L2_b.reshape(1, H),
        last_w.T, last_b.reshape(1, 1),
    )
    return out[:, 0]
